# baseline (speedup 1.0000x reference)
import functools

import jax
import jax.numpy as jnp
from jax import lax
from jax.experimental import pallas as pl
from jax.experimental.pallas import tpu as pltpu
from jax.experimental.pallas import tpu_sc as plsc

VOCAB = 100000
HIDDEN = 1024
MAX_POS = 4096
BATCH = 4
SEQ = 4096
EPS = 1e-12

NW = 32
C = 16
SPW = SEQ // NW
NJ = SPW // C
L = 16
HV = HIDDEN // L


def _rsqrt(v):
    half = v * 0.5
    i = plsc.bitcast(v, jnp.int32)
    i = jnp.int32(0x5F3759DF) - (i >> 1)
    y = plsc.bitcast(i, jnp.float32)
    y = y * (1.5 - half * y * y)
    y = y * (1.5 - half * y * y)
    y = y * (1.5 - half * y * y)
    return y


def _sc_embed_kernel(idx_hbm, words_hbm, pos_hbm, lnw_hbm, out_hbm,
                     idx_v, pbuf, wbuf, lnw_v, sem):
    wid = lax.axis_index("s") * 2 + lax.axis_index("c")

    pltpu.sync_copy(lnw_hbm, lnw_v)
    pltpu.sync_copy(idx_hbm.at[wid], idx_v)

    def jbody(j, carry):
        s0 = wid * SPW + j * C
        pltpu.sync_copy(pos_hbm.at[pl.ds(s0, C)], pbuf)
        for b in range(BATCH):
            pltpu.async_copy(words_hbm.at[idx_v.at[j * BATCH + b]], wbuf,
                             sem).wait()

            def tbody(t, tc):
                acc = jnp.zeros((L,), jnp.float32)
                for h in range(HV):
                    w = wbuf[t, pl.ds(h * L, L)]
                    p = pbuf[t, pl.ds(h * L, L)]
                    x = w + p
                    wbuf[t, pl.ds(h * L, L)] = x
                    acc = acc + x * x
                r = jnp.sum(acc) * (1.0 / HIDDEN) + EPS
                s = _rsqrt(jnp.broadcast_to(r, (L,)))
                for h in range(HV):
                    x = wbuf[t, pl.ds(h * L, L)]
                    wbuf[t, pl.ds(h * L, L)] = x * s * lnw_v[pl.ds(h * L, L)]
                return tc

            lax.fori_loop(0, C, tbody, 0)
            pltpu.sync_copy(wbuf, out_hbm.at[b, pl.ds(s0, C)])
        return carry

    lax.fori_loop(0, NJ, jbody, 0)


@jax.jit
def _sc_embed(idx, words, pos, lnw):
    mesh = plsc.VectorSubcoreMesh(core_axis_name="c", subcore_axis_name="s")
    f = functools.partial(
        pl.kernel,
        mesh=mesh,
        out_type=jax.ShapeDtypeStruct((BATCH, SEQ, HIDDEN), jnp.float32),
        scratch_types=[
            pltpu.VMEM((NJ * BATCH, C), jnp.int32),
            pltpu.VMEM((C, HIDDEN), jnp.float32),
            pltpu.VMEM((C, HIDDEN), jnp.float32),
            pltpu.VMEM((HIDDEN,), jnp.float32),
            pltpu.SemaphoreType.DMA,
        ],
        compiler_params=pltpu.CompilerParams(needs_layout_passes=False),
    )(_sc_embed_kernel)
    return f(idx, words, pos, lnw)


def kernel(input_ids, word_embeddings, position_embeddings, ln_weight):
    ids = input_ids.astype(jnp.int32)
    idx = ids.reshape(BATCH, NW, NJ, C).transpose(1, 2, 0, 3)
    idx = idx.reshape(NW, NJ * BATCH, C)
    return _sc_embed(idx, word_embeddings, position_embeddings, ln_weight)
